# Initial kernel scaffold; baseline (speedup 1.0000x reference)
import functools

import jax
import jax.numpy as jnp
from jax import lax
from jax.experimental import pallas as pl
from jax.experimental.pallas import tpu as pltpu
from jax.experimental.pallas import tpu_sc as plsc

N = 10000
E = 320000
D = 128
NC = 2
NS = 16
NW = NC * NS
EPT = E // NW
C = 128
NCH = -(-EPT // C)
PADT = NCH * C
ACC_H = 626 * NS
ZROWS = 626

_mesh = plsc.VectorSubcoreMesh(core_axis_name="c", subcore_axis_name="s")


@functools.partial(
    pl.kernel,
    out_type=jax.ShapeDtypeStruct((NC, N, 16), jnp.float32),
    mesh=_mesh,
    scratch_types=[
        pltpu.VMEM((NCH, 1, C), jnp.int32),
        pltpu.VMEM((C, 16), jnp.float32),
        pltpu.VMEM_SHARED((ACC_H, 16), jnp.float32),
    ],
)
def _deg_kernel(dst_hbm, out_hbm, dst_v, buf_v, acc):
    cid = lax.axis_index("c")
    sid = lax.axis_index("s")
    wid = sid * NC + cid

    @pl.loop(0, C)
    def _(i):
        buf_v[i, :] = jnp.zeros((16,), jnp.float32)

    zbase = sid * ZROWS
    for k in range(ZROWS // C):
        pltpu.sync_copy(buf_v, acc.at[pl.ds(zbase + k * C, C)])
    rem = ZROWS % C
    pltpu.sync_copy(buf_v.at[pl.ds(0, rem)],
                    acc.at[pl.ds(zbase + (ZROWS // C) * C, rem)])
    plsc.subcore_barrier()

    @pl.loop(0, C)
    def _(i):
        buf_v[i, :] = jnp.ones((16,), jnp.float32)

    pltpu.sync_copy(dst_hbm.at[wid], dst_v)

    @pl.loop(0, NCH)
    def _(j):
        pltpu.sync_copy(buf_v, acc.at[dst_v.at[j]], add=True)

    plsc.subcore_barrier()
    base = sid * (N // NS)
    pltpu.sync_copy(acc.at[pl.ds(base, N // NS)],
                    out_hbm.at[cid].at[pl.ds(base, N // NS)])


@functools.partial(
    pl.kernel,
    out_type=jax.ShapeDtypeStruct((NC, N, D), jnp.float32),
    mesh=_mesh,
    scratch_types=[
        pltpu.VMEM((NCH, C), jnp.int32),
        pltpu.VMEM((NCH, 1, C), jnp.int32),
        pltpu.VMEM((C, D), jnp.float32),
        pltpu.VMEM_SHARED((ACC_H, D), jnp.float32),
    ],
)
def _hop_kernel(g_hbm, src_hbm, dst_hbm, out_hbm, src_v, dst_v, rows_v, acc):
    cid = lax.axis_index("c")
    sid = lax.axis_index("s")
    wid = sid * NC + cid

    @pl.loop(0, C)
    def _(i):
        for k in range(D // 16):
            rows_v[i, pl.ds(k * 16, 16)] = jnp.zeros((16,), jnp.float32)

    zbase = sid * ZROWS
    for k in range(ZROWS // C):
        pltpu.sync_copy(rows_v, acc.at[pl.ds(zbase + k * C, C)])
    rem = ZROWS % C
    pltpu.sync_copy(rows_v.at[pl.ds(0, rem)],
                    acc.at[pl.ds(zbase + (ZROWS // C) * C, rem)])
    plsc.subcore_barrier()

    pltpu.sync_copy(src_hbm.at[wid], src_v)
    pltpu.sync_copy(dst_hbm.at[wid], dst_v)

    @pl.loop(0, NCH)
    def _(j):
        pltpu.sync_copy(g_hbm.at[src_v.at[j]], rows_v)
        pltpu.sync_copy(rows_v, acc.at[dst_v.at[j]], add=True)

    plsc.subcore_barrier()
    base = sid * (N // NS)
    pltpu.sync_copy(acc.at[pl.ds(base, N // NS)],
                    out_hbm.at[cid].at[pl.ds(base, N // NS)])


_BLK = 1000
_GRID = N // _BLK


def _deg_of(dr):
    deg = dr[0][:, 0:1] + dr[1][:, 0:1]
    return jnp.maximum(deg, 1.0)


def _prescale_body(f_ref, dr_ref, g_ref):
    dr = dr_ref[...]
    g_ref[...] = f_ref[...] * lax.rsqrt(_deg_of(dr))


def _mid_body(a_ref, dr_ref, g_ref):
    a = a_ref[...]
    dr = dr_ref[...]
    g_ref[...] = (a[0] + a[1]) / _deg_of(dr)


def _final_body(a_ref, dr_ref, w_ref, o_ref):
    a = a_ref[...]
    dr = dr_ref[...]
    h = (a[0] + a[1]) * lax.rsqrt(_deg_of(dr))
    o_ref[...] = lax.dot_general(
        h, w_ref[...], (((1,), (1,)), ((), ())),
        preferred_element_type=jnp.float32,
        precision=lax.Precision.HIGHEST,
    )


_feat_spec = pl.BlockSpec((_BLK, D), lambda i: (i, 0))
_deg_spec = pl.BlockSpec((NC, _BLK, 16), lambda i: (0, i, 0))
_acc_spec = pl.BlockSpec((NC, _BLK, D), lambda i: (0, i, 0))
_w_spec = pl.BlockSpec((D, D), lambda i: (0, 0))
_out_struct = jax.ShapeDtypeStruct((N, D), jnp.float32)

_prescale = pl.pallas_call(
    _prescale_body, grid=(_GRID,),
    in_specs=[_feat_spec, _deg_spec], out_specs=_feat_spec,
    out_shape=_out_struct)

_mid = pl.pallas_call(
    _mid_body, grid=(_GRID,),
    in_specs=[_acc_spec, _deg_spec], out_specs=_feat_spec,
    out_shape=_out_struct)

_final = pl.pallas_call(
    _final_body, grid=(_GRID,),
    in_specs=[_acc_spec, _deg_spec, _w_spec], out_specs=_feat_spec,
    out_shape=_out_struct)


def kernel(features, edge_index, W):
    src = edge_index[0].astype(jnp.int32).reshape(NW, EPT)
    dst = edge_index[1].astype(jnp.int32).reshape(NW, EPT)
    pad = PADT - EPT
    src3 = jnp.pad(src, ((0, 0), (0, pad))).reshape(NW, NCH, C)
    dst3 = jnp.pad(dst, ((0, 0), (0, pad)),
                   constant_values=N).reshape(NW, NCH, 1, C)

    degrep = _deg_kernel(dst3)
    g0 = _prescale(features, degrep)
    acc1 = _hop_kernel(g0, src3, dst3)
    g1 = _mid(acc1, degrep)
    acc2 = _hop_kernel(g1, src3, dst3)
    return _final(acc2, degrep, W)


# speedup vs baseline: 4.6864x; 4.6864x over previous
import functools

import jax
import jax.numpy as jnp
from jax import lax
from jax.experimental import pallas as pl
from jax.experimental.pallas import tpu as pltpu
from jax.experimental.pallas import tpu_sc as plsc

N = 10000
E = 320000
D = 128
NC = 2
NS = 16
NW = NC * NS
EPT = E // NW
C = 128
NCH = -(-EPT // C)
PADT = NCH * C
ZROWS = 632
ACC_H = ZROWS * NS

def _deg_body(dst_hbm, out_hbm, dst_v, buf_v, acc):
    cid = lax.axis_index("c")
    sid = lax.axis_index("s")
    wid = sid * NC + cid

    @pl.loop(0, C)
    def _(i):
        for k in range(D // 16):
            buf_v[i, pl.ds(k * 16, 16)] = jnp.zeros((16,), jnp.float32)

    zbase = sid * ZROWS
    for k in range(ZROWS // C):
        pltpu.sync_copy(buf_v, acc.at[pl.ds(zbase + k * C, C)])
    rem = ZROWS % C
    pltpu.sync_copy(buf_v.at[pl.ds(0, rem)],
                    acc.at[pl.ds(zbase + (ZROWS // C) * C, rem)])
    plsc.subcore_barrier()

    @pl.loop(0, C)
    def _(i):
        for k in range(D // 16):
            buf_v[i, pl.ds(k * 16, 16)] = jnp.ones((16,), jnp.float32)

    pltpu.sync_copy(dst_hbm.at[wid], dst_v)

    @pl.loop(0, NCH)
    def _(j):
        pltpu.sync_copy(buf_v, acc.at[dst_v.at[j]], add=True)

    plsc.subcore_barrier()
    base = sid * ZROWS
    pltpu.sync_copy(acc.at[pl.ds(base, ZROWS)],
                    out_hbm.at[cid].at[pl.ds(base, ZROWS)])


def _hop_body(g_hbm, src_hbm, dst_hbm, out_hbm, src_v, dst_v, rows_v, acc):
    cid = lax.axis_index("c")
    sid = lax.axis_index("s")
    wid = sid * NC + cid

    @pl.loop(0, C)
    def _(i):
        for k in range(D // 16):
            rows_v[i, pl.ds(k * 16, 16)] = jnp.zeros((16,), jnp.float32)

    zbase = sid * ZROWS
    for k in range(ZROWS // C):
        pltpu.sync_copy(rows_v, acc.at[pl.ds(zbase + k * C, C)])
    rem = ZROWS % C
    pltpu.sync_copy(rows_v.at[pl.ds(0, rem)],
                    acc.at[pl.ds(zbase + (ZROWS // C) * C, rem)])
    plsc.subcore_barrier()

    pltpu.sync_copy(src_hbm.at[wid], src_v)
    pltpu.sync_copy(dst_hbm.at[wid], dst_v)

    @pl.loop(0, NCH)
    def _(j):
        pltpu.sync_copy(g_hbm.at[src_v.at[j]], rows_v)
        pltpu.sync_copy(rows_v, acc.at[dst_v.at[j]], add=True)

    plsc.subcore_barrier()
    base = sid * ZROWS
    pltpu.sync_copy(acc.at[pl.ds(base, ZROWS)],
                    out_hbm.at[cid].at[pl.ds(base, ZROWS)])


@functools.cache
def _sc_kernels():
    mesh = plsc.VectorSubcoreMesh(core_axis_name="c", subcore_axis_name="s")
    deg_kernel = pl.kernel(
        _deg_body,
        out_type=jax.ShapeDtypeStruct((NC, ACC_H, D), jnp.float32),
        mesh=mesh,
        scratch_types=[
            pltpu.VMEM((NCH, C), jnp.int32),
            pltpu.VMEM((C, D), jnp.float32),
            pltpu.VMEM_SHARED((ACC_H, D), jnp.float32),
        ],
    )
    hop_kernel = pl.kernel(
        _hop_body,
        out_type=jax.ShapeDtypeStruct((NC, ACC_H, D), jnp.float32),
        mesh=mesh,
        scratch_types=[
            pltpu.VMEM((NCH, C), jnp.int32),
            pltpu.VMEM((NCH, C), jnp.int32),
            pltpu.VMEM((C, D), jnp.float32),
            pltpu.VMEM_SHARED((ACC_H, D), jnp.float32),
        ],
    )
    return deg_kernel, hop_kernel


_BLK = 1000
_GRID = N // _BLK


def _deg_of(dr):
    deg = dr[0][:, 0:1] + dr[1][:, 0:1]
    return jnp.maximum(deg, 1.0)


def _prescale_body(f_ref, dr_ref, g_ref):
    dr = dr_ref[...]
    g_ref[...] = f_ref[...] * lax.rsqrt(_deg_of(dr))


def _mid_body(a_ref, dr_ref, g_ref):
    a = a_ref[...]
    dr = dr_ref[...]
    g_ref[...] = (a[0] + a[1]) / _deg_of(dr)


def _final_body(a_ref, dr_ref, w_ref, o_ref):
    a = a_ref[...]
    dr = dr_ref[...]
    h = (a[0] + a[1]) * lax.rsqrt(_deg_of(dr))
    o_ref[...] = lax.dot_general(
        h, w_ref[...], (((1,), (1,)), ((), ())),
        preferred_element_type=jnp.float32,
        precision=lax.Precision.HIGHEST,
    )


_feat_spec = pl.BlockSpec((_BLK, D), lambda i: (i, 0))
_deg_spec = pl.BlockSpec((NC, _BLK, D), lambda i: (0, i, 0))
_acc_spec = pl.BlockSpec((NC, _BLK, D), lambda i: (0, i, 0))
_w_spec = pl.BlockSpec((D, D), lambda i: (0, 0))
_out_struct = jax.ShapeDtypeStruct((N, D), jnp.float32)

_prescale = pl.pallas_call(
    _prescale_body, grid=(_GRID,),
    in_specs=[_feat_spec, _deg_spec], out_specs=_feat_spec,
    out_shape=_out_struct)

_mid = pl.pallas_call(
    _mid_body, grid=(_GRID,),
    in_specs=[_acc_spec, _deg_spec], out_specs=_feat_spec,
    out_shape=_out_struct)

_final = pl.pallas_call(
    _final_body, grid=(_GRID,),
    in_specs=[_acc_spec, _deg_spec, _w_spec], out_specs=_feat_spec,
    out_shape=_out_struct)


def kernel(features, edge_index, W):
    src = edge_index[0].astype(jnp.int32).reshape(NW, EPT)
    dst = edge_index[1].astype(jnp.int32).reshape(NW, EPT)
    pad = PADT - EPT
    src3 = jnp.pad(src, ((0, 0), (0, pad))).reshape(NW, NCH, C)
    dst3 = jnp.pad(dst, ((0, 0), (0, pad)),
                   constant_values=N).reshape(NW, NCH, C)

    deg_kernel, hop_kernel = _sc_kernels()
    degrep = deg_kernel(dst3)
    g0 = _prescale(features, degrep)
    acc1 = hop_kernel(g0, src3, dst3)
    g1 = _mid(acc1, degrep)
    acc2 = hop_kernel(g1, src3, dst3)
    return _final(acc2, degrep, W)
